# speedup vs baseline: 1.2515x; 1.2515x over previous
import functools

import jax
import jax.numpy as jnp
from jax.experimental import pallas as pl
from jax.experimental.pallas import tpu as pltpu

VX, VY, VZ = 0.32, 0.32, 6.0
XMIN, YMIN, ZMIN = -74.88, -74.88, -2.0
NX, NY, NZ = 468, 468, 1
CANVAS = NZ * NY * NX
X_OFF = VX / 2 + XMIN
Y_OFF = VY / 2 + YMIN
Z_OFF = VZ / 2 + ZMIN

N = 160000
BLK = 3200


def _pf1_body(points_ref, pmean_ref, cxy_ref, W1_ref, b1_ref, out_ref):
    pts = points_ref[...]
    xyz = pts[:, :3]
    f_cluster = xyz - pmean_ref[...]
    cxy = cxy_ref[...]
    cx = cxy[:, 0] * VX + X_OFF
    cy = cxy[:, 1] * VY + Y_OFF
    cz = cxy[:, 2] * VZ + Z_OFF
    f_center = jnp.stack([pts[:, 0] - cx, pts[:, 1] - cy, pts[:, 2] - cz], axis=1)
    feats = jnp.concatenate([pts, f_cluster, f_center], axis=1)
    acc = jnp.dot(feats, W1_ref[...], preferred_element_type=jnp.float32)
    out_ref[...] = jnp.maximum(acc + b1_ref[...], 0.0)


def _pf2_body(pf1_ref, vg_ref, W2a_ref, W2b_ref, b2_ref, out_ref):
    acc = jnp.dot(pf1_ref[...], W2a_ref[...], preferred_element_type=jnp.float32)
    acc += jnp.dot(vg_ref[...], W2b_ref[...], preferred_element_type=jnp.float32)
    out_ref[...] = jnp.maximum(acc + b2_ref[...], 0.0)


@jax.jit
def kernel(points, W1, b1, W2, b2):
    cx = jnp.clip(jnp.floor((points[:, 0] - XMIN) / VX).astype(jnp.int32), 0, NX - 1)
    cy = jnp.clip(jnp.floor((points[:, 1] - YMIN) / VY).astype(jnp.int32), 0, NY - 1)
    cz = jnp.clip(jnp.floor((points[:, 2] - ZMIN) / VZ).astype(jnp.int32), 0, NZ - 1)
    vidx = cz * (NY * NX) + cy * NX + cx

    sums = jax.ops.segment_sum(points, vidx, num_segments=CANVAS)
    cnt = jax.ops.segment_sum(jnp.ones((N, 1), points.dtype), vidx, num_segments=CANVAS)
    voxel_mean = sums / jnp.maximum(cnt, 1.0)
    pmean = voxel_mean[vidx][:, :3]
    cxy = jnp.stack([cx, cy, cz], axis=1).astype(jnp.float32)

    grid = (N // BLK,)
    pf1 = pl.pallas_call(
        _pf1_body,
        grid=grid,
        in_specs=[
            pl.BlockSpec((BLK, 5), lambda i: (i, 0)),
            pl.BlockSpec((BLK, 3), lambda i: (i, 0)),
            pl.BlockSpec((BLK, 3), lambda i: (i, 0)),
            pl.BlockSpec((11, 64), lambda i: (0, 0)),
            pl.BlockSpec((64,), lambda i: (0,)),
        ],
        out_specs=pl.BlockSpec((BLK, 64), lambda i: (i, 0)),
        out_shape=jax.ShapeDtypeStruct((N, 64), jnp.float32),
    )(points, pmean, cxy, W1, b1)

    occ = cnt > 0
    vf1 = jnp.where(occ, jax.ops.segment_max(pf1, vidx, num_segments=CANVAS), 0.0)
    vg = vf1[vidx]

    pf2 = pl.pallas_call(
        _pf2_body,
        grid=grid,
        in_specs=[
            pl.BlockSpec((BLK, 64), lambda i: (i, 0)),
            pl.BlockSpec((BLK, 64), lambda i: (i, 0)),
            pl.BlockSpec((64, 64), lambda i: (0, 0)),
            pl.BlockSpec((64, 64), lambda i: (0, 0)),
            pl.BlockSpec((64,), lambda i: (0,)),
        ],
        out_specs=pl.BlockSpec((BLK, 64), lambda i: (i, 0)),
        out_shape=jax.ShapeDtypeStruct((N, 64), jnp.float32),
    )(pf1, vg, W2[:64], W2[64:], b2)

    voxel_feats = jnp.where(occ, jax.ops.segment_max(pf2, vidx, num_segments=CANVAS), 0.0)
    lin = jnp.arange(CANVAS, dtype=jnp.int32)
    voxel_coors = jnp.stack(
        [jnp.zeros_like(lin), lin // (NY * NX), (lin // NX) % NY, lin % NX], axis=1
    )
    return voxel_feats, voxel_coors
